# Initial kernel scaffold; baseline (speedup 1.0000x reference)
import jax
import jax.numpy as jnp
from jax.experimental import pallas as pl


def kernel(inputs, targets):
    raise NotImplementedError("write your pallas kernel here")


# speedup vs baseline: 1.0005x; 1.0005x over previous
import jax
import jax.numpy as jnp
from jax.experimental import pallas as pl
from jax.experimental.pallas import tpu as pltpu


def _kl_sum_kernel(x_ref, t_ref, o_ref):
    i = pl.program_id(0)
    t = t_ref[...]
    x = x_ref[...]
    safe_t = jnp.where(t > 0, t, 1.0)
    kl = t * jnp.log(safe_t) - t * x
    s = jnp.sum(kl)

    @pl.when(i == 0)
    def _init():
        o_ref[0, 0] = 0.0

    o_ref[0, 0] += s


def kernel(inputs, targets):
    rows, cols = inputs.shape
    block_rows = 512
    grid = rows // block_rows

    out = pl.pallas_call(
        _kl_sum_kernel,
        grid=(grid,),
        in_specs=[
            pl.BlockSpec((block_rows, cols), lambda i: (i, 0)),
            pl.BlockSpec((block_rows, cols), lambda i: (i, 0)),
        ],
        out_specs=pl.BlockSpec(
            (1, 1), lambda i: (0, 0), memory_space=pltpu.SMEM
        ),
        out_shape=jax.ShapeDtypeStruct((1, 1), jnp.float32),
        compiler_params=pltpu.CompilerParams(
            dimension_semantics=("arbitrary",),
        ),
    )(inputs, targets)
    return (out[0, 0] / (rows * cols)).astype(jnp.float32)
